# baseline (device time: 234610 ns/iter reference)
import jax
import jax.numpy as jnp
from jax import lax
from jax.experimental import pallas as pl
from jax.experimental.pallas import tpu as pltpu

N_DEV = 32
M = 1536
N = 1536
CHUNK = M // N_DEV
N_STEPS = N_DEV - 1


def _silu_f32(v):
    return v * (1.0 / (1.0 + jnp.exp(-v)))


def kernel(A, B):
    def body(a_ref, b_ref, out_ref, z_ref, comm_ref, send_sems, recv_sems, exit_sem):
        my = lax.axis_index("i")
        left = lax.rem(my + N_DEV - 1, N_DEV)
        right = lax.rem(my + 1, N_DEV)

        barrier = pltpu.get_barrier_semaphore()
        for nbr in (left, right):
            pl.semaphore_signal(
                barrier, inc=1, device_id=(nbr,),
                device_id_type=pl.DeviceIdType.MESH,
            )
        pl.semaphore_wait(barrier, 2)

        z_ref[...] = jnp.dot(
            a_ref[...].astype(jnp.bfloat16),
            b_ref[...].astype(jnp.bfloat16),
            preferred_element_type=jnp.float32,
        )

        def rows(c):
            return pl.ds(c * CHUNK, CHUNK)

        comm_ref[rows(my), :] = z_ref[rows(my), :].astype(jnp.bfloat16)

        for s in range(N_STEPS):
            c_send = lax.rem(my - s + N_DEV, N_DEV)
            c_recv = lax.rem(my - s - 1 + N_DEV, N_DEV)
            rdma = pltpu.make_async_remote_copy(
                src_ref=comm_ref.at[rows(c_send), :],
                dst_ref=comm_ref.at[rows(c_send), :],
                send_sem=send_sems.at[s],
                recv_sem=recv_sems.at[s],
                device_id=(right,),
                device_id_type=pl.DeviceIdType.MESH,
            )
            rdma.start()
            rdma.wait()
            comm_ref[rows(c_recv), :] = (
                comm_ref[rows(c_recv), :].astype(jnp.float32)
                + z_ref[rows(c_recv), :]
            ).astype(jnp.bfloat16)

        c_own = lax.rem(my + 1, N_DEV)
        out_ref[rows(c_own), :] = _silu_f32(
            comm_ref[rows(c_own), :].astype(jnp.float32)
        )

        for s in range(N_STEPS):
            c_send = lax.rem(my + 1 - s + N_DEV, N_DEV)
            c_recv = lax.rem(my - s + N_DEV, N_DEV)
            rdma = pltpu.make_async_remote_copy(
                src_ref=comm_ref.at[rows(c_send), :],
                dst_ref=comm_ref.at[rows(c_send), :],
                send_sem=send_sems.at[N_STEPS + s],
                recv_sem=recv_sems.at[N_STEPS + s],
                device_id=(right,),
                device_id_type=pl.DeviceIdType.MESH,
            )
            rdma.start()
            rdma.wait()
            out_ref[rows(c_recv), :] = _silu_f32(
                comm_ref[rows(c_recv), :].astype(jnp.float32)
            )

        for nbr in (left, right):
            pl.semaphore_signal(
                exit_sem, inc=1, device_id=(nbr,),
                device_id_type=pl.DeviceIdType.MESH,
            )
        pl.semaphore_wait(exit_sem, 2)

    return pl.pallas_call(
        body,
        out_shape=jax.ShapeDtypeStruct((M, N), jnp.float32),
        in_specs=[
            pl.BlockSpec(memory_space=pltpu.VMEM),
            pl.BlockSpec(memory_space=pltpu.VMEM),
        ],
        out_specs=pl.BlockSpec(memory_space=pltpu.VMEM),
        scratch_shapes=[
            pltpu.VMEM((M, N), jnp.float32),
            pltpu.VMEM((M, N), jnp.bfloat16),
            pltpu.SemaphoreType.DMA((2 * N_STEPS,)),
            pltpu.SemaphoreType.DMA((2 * N_STEPS,)),
            pltpu.SemaphoreType.REGULAR,
        ],
        compiler_params=pltpu.CompilerParams(collective_id=0),
    )(A, B)


# device time: 147570 ns/iter; 1.5898x vs baseline; 1.5898x over previous
import jax
import jax.numpy as jnp
from jax import lax
from jax.experimental import pallas as pl
from jax.experimental.pallas import tpu as pltpu

N_DEV = 32
M = 1536
N = 1536

MASKS = (1, 3, 4, 8, 16)
SHIFTS = (0, 1, 2, 3, 4)
HALVES = (768, 384, 192, 96, 48)
OFFS = (0, 768, 1152, 1344, 1440)


def _silu_f32(v):
    return v * (1.0 / (1.0 + jnp.exp(-v)))


def kernel(A, B):
    def body(a_ref, b_ref, out_ref, z_ref, zb_ref, rs_ref, ag_ref,
             send_sems, recv_sems, exit_sem):
        my = lax.axis_index("i")

        def keep_bit(s):
            if s == 0:
                return (my ^ (my >> 1)) & 1
            return (my >> SHIFTS[s]) & 1

        partners = [my ^ m for m in MASKS]

        barrier = pltpu.get_barrier_semaphore()
        for j in partners:
            pl.semaphore_signal(
                barrier, inc=1, device_id=(j,),
                device_id_type=pl.DeviceIdType.MESH,
            )
        pl.semaphore_wait(barrier, len(partners))

        def rdma(s, src_lo, dst_lo, half, sem_idx):
            return pltpu.make_async_remote_copy(
                src_ref=src_lo,
                dst_ref=dst_lo,
                send_sem=send_sems.at[sem_idx],
                recv_sem=recv_sems.at[sem_idx],
                device_id=(partners[s],),
                device_id_type=pl.DeviceIdType.MESH,
            )

        def matmul(row_lo, n_rows):
            return jnp.dot(
                a_ref[pl.ds(row_lo, n_rows), :].astype(jnp.bfloat16),
                b_ref[...].astype(jnp.bfloat16),
                preferred_element_type=jnp.float32,
            )

        b0 = keep_bit(0)
        send_lo = (1 - b0) * HALVES[0]
        keep_lo = b0 * HALVES[0]
        z_ref[pl.ds(send_lo, HALVES[0]), :] = matmul(send_lo, HALVES[0])
        zb_ref[pl.ds(send_lo, HALVES[0]), :] = (
            z_ref[pl.ds(send_lo, HALVES[0]), :].astype(jnp.bfloat16)
        )
        op = rdma(
            0,
            zb_ref.at[pl.ds(send_lo, HALVES[0]), :],
            rs_ref.at[pl.ds(OFFS[0], HALVES[0]), :],
            HALVES[0],
            0,
        )
        op.start()
        z_ref[pl.ds(keep_lo, HALVES[0]), :] = matmul(keep_lo, HALVES[0])
        op.wait()
        z_ref[pl.ds(keep_lo, HALVES[0]), :] = (
            z_ref[pl.ds(keep_lo, HALVES[0]), :]
            + rs_ref[pl.ds(OFFS[0], HALVES[0]), :].astype(jnp.float32)
        )
        lo = keep_lo

        for s in range(1, 5):
            half = HALVES[s]
            b = keep_bit(s)
            send_lo = lo + (1 - b) * half
            keep_lo = lo + b * half
            zb_ref[pl.ds(send_lo, half), :] = (
                z_ref[pl.ds(send_lo, half), :].astype(jnp.bfloat16)
            )
            op = rdma(
                s,
                zb_ref.at[pl.ds(send_lo, half), :],
                rs_ref.at[pl.ds(OFFS[s], half), :],
                half,
                s,
            )
            op.start()
            op.wait()
            z_ref[pl.ds(keep_lo, half), :] = (
                z_ref[pl.ds(keep_lo, half), :]
                + rs_ref[pl.ds(OFFS[s], half), :].astype(jnp.float32)
            )
            lo = keep_lo

        own = _silu_f32(z_ref[pl.ds(lo, 48), :])
        out_ref[pl.ds(lo, 48), :] = own
        ag_ref[pl.ds(lo, 48), :] = own.astype(jnp.bfloat16)

        for k, s in enumerate(reversed(range(5))):
            half = HALVES[s]
            b = keep_bit(s)
            recv_lo = lo + (1 - 2 * b) * half
            op = rdma(
                s,
                ag_ref.at[pl.ds(lo, half), :],
                ag_ref.at[pl.ds(lo, half), :],
                half,
                5 + k,
            )
            op.start()
            op.wait()
            out_ref[pl.ds(recv_lo, half), :] = (
                ag_ref[pl.ds(recv_lo, half), :].astype(jnp.float32)
            )
            lo = lo - b * half

        for j in partners:
            pl.semaphore_signal(
                exit_sem, inc=1, device_id=(j,),
                device_id_type=pl.DeviceIdType.MESH,
            )
        pl.semaphore_wait(exit_sem, len(partners))

    return pl.pallas_call(
        body,
        out_shape=jax.ShapeDtypeStruct((M, N), jnp.float32),
        in_specs=[
            pl.BlockSpec(memory_space=pltpu.VMEM),
            pl.BlockSpec(memory_space=pltpu.VMEM),
        ],
        out_specs=pl.BlockSpec(memory_space=pltpu.VMEM),
        scratch_shapes=[
            pltpu.VMEM((M, N), jnp.float32),
            pltpu.VMEM((M, N), jnp.bfloat16),
            pltpu.VMEM((M, N), jnp.bfloat16),
            pltpu.VMEM((M, N), jnp.bfloat16),
            pltpu.SemaphoreType.DMA((10,)),
            pltpu.SemaphoreType.DMA((10,)),
            pltpu.SemaphoreType.REGULAR,
        ],
        compiler_params=pltpu.CompilerParams(collective_id=0),
    )(A, B)


# device time: 98976 ns/iter; 2.3704x vs baseline; 1.4910x over previous
import jax
import jax.numpy as jnp
from jax import lax
from jax.experimental import pallas as pl
from jax.experimental.pallas import tpu as pltpu

N_DEV = 32
M = 1536
N = 1536
COLS = N // 2

MASKS = (1, 3, 4, 8, 16)
HALVES = (768, 384, 192, 96, 48)
OFFS = (0, 768, 1152, 1344, 1440)

A_ORDER = (1, 3, 4, 8, 16)
B_ORDER = (8, 1, 3, 4, 16)


def _silu_f32(v):
    return v * (1.0 / (1.0 + jnp.exp(-v)))


def kernel(A, B):
    def body(a_ref, b_ref, out_ref, z_ref,
             zb_a, rs_a, ag_a, zb_b, rs_b, ag_b,
             send_sems, recv_sems, exit_sem):
        my = lax.axis_index("i")

        def phi(mask):
            if mask == 1:
                return (my ^ (my >> 1)) & 1
            shift = {3: 1, 4: 2, 8: 3, 16: 4}[mask]
            return (my >> shift) & 1

        barrier = pltpu.get_barrier_semaphore()
        for m in MASKS:
            pl.semaphore_signal(
                barrier, inc=1, device_id=(my ^ m,),
                device_id_type=pl.DeviceIdType.MESH,
            )
        pl.semaphore_wait(barrier, len(MASKS))

        streams = [
            dict(order=A_ORDER, col=0, zb=zb_a, rs=rs_a, ag=ag_a,
                 sem_base=0, lo=0),
            dict(order=B_ORDER, col=COLS, zb=zb_b, rs=rs_b, ag=ag_b,
                 sem_base=10, lo=0),
        ]

        def rdma(st, mask_stage, sem_idx, src, dst):
            k = st["sem_base"] + sem_idx
            return pltpu.make_async_remote_copy(
                src_ref=src,
                dst_ref=dst,
                send_sem=send_sems.at[k],
                recv_sem=recv_sems.at[k],
                device_id=(my ^ st["order"][mask_stage],),
                device_id_type=pl.DeviceIdType.MESH,
            )

        def matmul_quarter(row_lo, col_lo):
            z_ref[pl.ds(row_lo, HALVES[0]), pl.ds(col_lo, COLS)] = jnp.dot(
                a_ref[pl.ds(row_lo, HALVES[0]), :].astype(jnp.bfloat16),
                b_ref[:, pl.ds(col_lo, COLS)].astype(jnp.bfloat16),
                preferred_element_type=jnp.float32,
            )

        def start_rs(st, stage, send_lo, half):
            st["zb"][pl.ds(send_lo, half), :] = (
                z_ref[pl.ds(send_lo, half),
                      pl.ds(st["col"], COLS)].astype(jnp.bfloat16)
            )
            op = rdma(
                st, stage, stage,
                st["zb"].at[pl.ds(send_lo, half), :],
                st["rs"].at[pl.ds(OFFS[stage], half), :],
            )
            op.start()
            return op

        def finish_rs(st, stage, op, keep_lo, half):
            op.wait()
            z_ref[pl.ds(keep_lo, half), pl.ds(st["col"], COLS)] = (
                z_ref[pl.ds(keep_lo, half), pl.ds(st["col"], COLS)]
                + st["rs"][pl.ds(OFFS[stage], half), :].astype(jnp.float32)
            )
            st["lo"] = keep_lo

        ops = []
        for st in streams:
            b = phi(st["order"][0])
            send_lo = (1 - b) * HALVES[0]
            st["keep0"] = b * HALVES[0]
            matmul_quarter(send_lo, st["col"])
            ops.append(start_rs(st, 0, send_lo, HALVES[0]))
        for st in streams:
            matmul_quarter(st["keep0"], st["col"])
        for st, op in zip(streams, ops):
            finish_rs(st, 0, op, st["keep0"], HALVES[0])

        for s in range(1, 5):
            half = HALVES[s]
            ops = []
            for st in streams:
                b = phi(st["order"][s])
                send_lo = st["lo"] + (1 - b) * half
                st["keep"] = st["lo"] + b * half
                ops.append(start_rs(st, s, send_lo, half))
            for st, op in zip(streams, ops):
                finish_rs(st, s, op, st["keep"], half)

        for st in streams:
            own = _silu_f32(
                z_ref[pl.ds(st["lo"], 48), pl.ds(st["col"], COLS)]
            )
            out_ref[pl.ds(st["lo"], 48), pl.ds(st["col"], COLS)] = own
            st["ag"][pl.ds(st["lo"], 48), :] = own.astype(jnp.bfloat16)

        for k in range(5):
            s = 4 - k
            half = HALVES[s]
            ops = []
            for st in streams:
                b = phi(st["order"][s])
                st["recv_lo"] = st["lo"] + (1 - 2 * b) * half
                st["new_lo"] = st["lo"] - b * half
                op = rdma(
                    st, s, 5 + k,
                    st["ag"].at[pl.ds(st["lo"], half), :],
                    st["ag"].at[pl.ds(st["lo"], half), :],
                )
                op.start()
                ops.append(op)
            for st, op in zip(streams, ops):
                op.wait()
                out_ref[pl.ds(st["recv_lo"], half), pl.ds(st["col"], COLS)] = (
                    st["ag"][pl.ds(st["recv_lo"], half), :].astype(jnp.float32)
                )
                st["lo"] = st["new_lo"]

        for m in MASKS:
            pl.semaphore_signal(
                exit_sem, inc=1, device_id=(my ^ m,),
                device_id_type=pl.DeviceIdType.MESH,
            )
        pl.semaphore_wait(exit_sem, len(MASKS))

    return pl.pallas_call(
        body,
        out_shape=jax.ShapeDtypeStruct((M, N), jnp.float32),
        in_specs=[
            pl.BlockSpec(memory_space=pltpu.VMEM),
            pl.BlockSpec(memory_space=pltpu.VMEM),
        ],
        out_specs=pl.BlockSpec(memory_space=pltpu.VMEM),
        scratch_shapes=[
            pltpu.VMEM((M, N), jnp.float32),
            pltpu.VMEM((M, COLS), jnp.bfloat16),
            pltpu.VMEM((M, COLS), jnp.bfloat16),
            pltpu.VMEM((M, COLS), jnp.bfloat16),
            pltpu.VMEM((M, COLS), jnp.bfloat16),
            pltpu.VMEM((M, COLS), jnp.bfloat16),
            pltpu.VMEM((M, COLS), jnp.bfloat16),
            pltpu.SemaphoreType.DMA((20,)),
            pltpu.SemaphoreType.DMA((20,)),
            pltpu.SemaphoreType.REGULAR,
        ],
        compiler_params=pltpu.CompilerParams(collective_id=0),
    )(A, B)


# device time: 79170 ns/iter; 2.9634x vs baseline; 1.2502x over previous
import jax
import jax.numpy as jnp
from jax import lax
from jax.experimental import pallas as pl
from jax.experimental.pallas import tpu as pltpu

N_DEV = 32
M = 1536
N = 1536
N_STREAMS = 3
COLS = N // N_STREAMS

MASKS = (1, 3, 4, 8, 16)
HALVES = (768, 384, 192, 96, 48)
OFFS = (0, 768, 1152, 1344, 1440)

ORDERS = (
    (1, 3, 8, 4, 16),
    (8, 1, 3, 16, 4),
    (3, 8, 1, 4, 16),
)


def _silu_f32(v):
    return v * (1.0 / (1.0 + jnp.exp(-v)))


def kernel(A, B):
    def body(a_ref, b_ref, out_ref, z_ref,
             zb_a, rs_a, ag_a, zb_b, rs_b, ag_b, zb_c, rs_c, ag_c,
             send_sems, recv_sems, exit_sem):
        my = lax.axis_index("i")

        def phi(mask):
            if mask == 1:
                return (my ^ (my >> 1)) & 1
            shift = {3: 1, 4: 2, 8: 3, 16: 4}[mask]
            return (my >> shift) & 1

        barrier = pltpu.get_barrier_semaphore()
        for m in MASKS:
            pl.semaphore_signal(
                barrier, inc=1, device_id=(my ^ m,),
                device_id_type=pl.DeviceIdType.MESH,
            )
        pl.semaphore_wait(barrier, len(MASKS))

        bufs = [(zb_a, rs_a, ag_a), (zb_b, rs_b, ag_b), (zb_c, rs_c, ag_c)]
        streams = [
            dict(order=ORDERS[t], col=t * COLS, zb=bufs[t][0],
                 rs=bufs[t][1], ag=bufs[t][2], sem_base=10 * t, lo=0)
            for t in range(N_STREAMS)
        ]

        def rdma(st, mask_stage, sem_idx, src, dst):
            k = st["sem_base"] + sem_idx
            return pltpu.make_async_remote_copy(
                src_ref=src,
                dst_ref=dst,
                send_sem=send_sems.at[k],
                recv_sem=recv_sems.at[k],
                device_id=(my ^ st["order"][mask_stage],),
                device_id_type=pl.DeviceIdType.MESH,
            )

        def matmul_piece(row_lo, col_lo):
            z_ref[pl.ds(row_lo, HALVES[0]), pl.ds(col_lo, COLS)] = jnp.dot(
                a_ref[pl.ds(row_lo, HALVES[0]), :].astype(jnp.bfloat16),
                b_ref[:, pl.ds(col_lo, COLS)].astype(jnp.bfloat16),
                preferred_element_type=jnp.float32,
            )

        def start_rs(st, stage, send_lo, half):
            st["zb"][pl.ds(send_lo, half), :] = (
                z_ref[pl.ds(send_lo, half),
                      pl.ds(st["col"], COLS)].astype(jnp.bfloat16)
            )
            op = rdma(
                st, stage, stage,
                st["zb"].at[pl.ds(send_lo, half), :],
                st["rs"].at[pl.ds(OFFS[stage], half), :],
            )
            op.start()
            return op

        def finish_rs(st, stage, op, keep_lo, half):
            op.wait()
            z_ref[pl.ds(keep_lo, half), pl.ds(st["col"], COLS)] = (
                z_ref[pl.ds(keep_lo, half), pl.ds(st["col"], COLS)]
                + st["rs"][pl.ds(OFFS[stage], half), :].astype(jnp.float32)
            )
            st["lo"] = keep_lo

        ops = []
        for st in streams:
            b = phi(st["order"][0])
            send_lo = (1 - b) * HALVES[0]
            st["keep0"] = b * HALVES[0]
            matmul_piece(send_lo, st["col"])
            ops.append(start_rs(st, 0, send_lo, HALVES[0]))
        for st in streams:
            matmul_piece(st["keep0"], st["col"])
        for st, op in zip(streams, ops):
            finish_rs(st, 0, op, st["keep0"], HALVES[0])

        for s in range(1, 5):
            half = HALVES[s]
            ops = []
            for st in streams:
                b = phi(st["order"][s])
                send_lo = st["lo"] + (1 - b) * half
                st["keep"] = st["lo"] + b * half
                ops.append(start_rs(st, s, send_lo, half))
            for st, op in zip(streams, ops):
                finish_rs(st, s, op, st["keep"], half)

        for st in streams:
            own = _silu_f32(
                z_ref[pl.ds(st["lo"], 48), pl.ds(st["col"], COLS)]
            )
            out_ref[pl.ds(st["lo"], 48), pl.ds(st["col"], COLS)] = own
            st["ag"][pl.ds(st["lo"], 48), :] = own.astype(jnp.bfloat16)

        for k in range(5):
            s = 4 - k
            half = HALVES[s]
            ops = []
            for st in streams:
                b = phi(st["order"][s])
                st["recv_lo"] = st["lo"] + (1 - 2 * b) * half
                st["new_lo"] = st["lo"] - b * half
                op = rdma(
                    st, s, 5 + k,
                    st["ag"].at[pl.ds(st["lo"], half), :],
                    st["ag"].at[pl.ds(st["lo"], half), :],
                )
                op.start()
                ops.append(op)
            for st, op in zip(streams, ops):
                op.wait()
                out_ref[pl.ds(st["recv_lo"], half), pl.ds(st["col"], COLS)] = (
                    st["ag"][pl.ds(st["recv_lo"], half), :].astype(jnp.float32)
                )
                st["lo"] = st["new_lo"]

        for m in MASKS:
            pl.semaphore_signal(
                exit_sem, inc=1, device_id=(my ^ m,),
                device_id_type=pl.DeviceIdType.MESH,
            )
        pl.semaphore_wait(exit_sem, len(MASKS))

    stream_scratch = []
    for _ in range(N_STREAMS):
        stream_scratch += [
            pltpu.VMEM((M, COLS), jnp.bfloat16),
            pltpu.VMEM((M, COLS), jnp.bfloat16),
            pltpu.VMEM((M, COLS), jnp.bfloat16),
        ]

    return pl.pallas_call(
        body,
        out_shape=jax.ShapeDtypeStruct((M, N), jnp.float32),
        in_specs=[
            pl.BlockSpec(memory_space=pltpu.VMEM),
            pl.BlockSpec(memory_space=pltpu.VMEM),
        ],
        out_specs=pl.BlockSpec(memory_space=pltpu.VMEM),
        scratch_shapes=[
            pltpu.VMEM((M, N), jnp.float32),
            *stream_scratch,
            pltpu.SemaphoreType.DMA((10 * N_STREAMS,)),
            pltpu.SemaphoreType.DMA((10 * N_STREAMS,)),
            pltpu.SemaphoreType.REGULAR,
        ],
        compiler_params=pltpu.CompilerParams(collective_id=0),
    )(A, B)


# device time: 73569 ns/iter; 3.1890x vs baseline; 1.0761x over previous
import jax
import jax.numpy as jnp
from jax import lax
from jax.experimental import pallas as pl
from jax.experimental.pallas import tpu as pltpu

N_DEV = 32
M = 1536
N = 1536
N_STREAMS = 3
COLS = N // N_STREAMS

MASKS = (1, 3, 4, 8, 16)
HALVES = (768, 384, 192, 96, 48)
OFFS = (0, 768, 1152, 1344, 1440)

ORDERS = (
    (1, 3, 8, 4, 16),
    (8, 1, 3, 16, 4),
    (3, 8, 1, 4, 16),
)


def _silu_f32(v):
    return v * (1.0 / (1.0 + jnp.exp(-v)))


def kernel(A, B):
    def body(a_ref, b_ref, out_ref, z_ref,
             zb_a, rs_a, ag_a, zb_b, rs_b, ag_b, zb_c, rs_c, ag_c,
             send_sems, recv_sems, exit_sem):
        my = lax.axis_index("i")

        def phi(mask):
            if mask == 1:
                return (my ^ (my >> 1)) & 1
            shift = {3: 1, 4: 2, 8: 3, 16: 4}[mask]
            return (my >> shift) & 1

        barrier = pltpu.get_barrier_semaphore()
        for m in MASKS:
            pl.semaphore_signal(
                barrier, inc=1, device_id=(my ^ m,),
                device_id_type=pl.DeviceIdType.MESH,
            )
        pl.semaphore_wait(barrier, len(MASKS))

        bufs = [(zb_a, rs_a, ag_a), (zb_b, rs_b, ag_b), (zb_c, rs_c, ag_c)]
        streams = [
            dict(order=ORDERS[t], col=t * COLS, zb=bufs[t][0],
                 rs=bufs[t][1], ag=bufs[t][2], sem_base=10 * t, lo=0)
            for t in range(N_STREAMS)
        ]

        def rdma(st, mask_stage, sem_idx, src, dst):
            k = st["sem_base"] + sem_idx
            return pltpu.make_async_remote_copy(
                src_ref=src,
                dst_ref=dst,
                send_sem=send_sems.at[k],
                recv_sem=recv_sems.at[k],
                device_id=(my ^ st["order"][mask_stage],),
                device_id_type=pl.DeviceIdType.MESH,
            )

        def matmul_piece(row_lo, col_lo):
            z_ref[pl.ds(row_lo, HALVES[0]), pl.ds(col_lo, COLS)] = jnp.dot(
                a_ref[pl.ds(row_lo, HALVES[0]), :].astype(jnp.bfloat16),
                b_ref[:, pl.ds(col_lo, COLS)].astype(jnp.bfloat16),
                preferred_element_type=jnp.float32,
            )

        def start_rs(st, stage, send_lo, half):
            st["zb"][pl.ds(send_lo, half), :] = (
                z_ref[pl.ds(send_lo, half),
                      pl.ds(st["col"], COLS)].astype(jnp.bfloat16)
            )
            op = rdma(
                st, stage, stage,
                st["zb"].at[pl.ds(send_lo, half), :],
                st["rs"].at[pl.ds(OFFS[stage], half), :],
            )
            op.start()
            return op

        def add_recv(st, z_lo, stage_off, n_rows):
            z_ref[pl.ds(z_lo, n_rows), pl.ds(st["col"], COLS)] = (
                z_ref[pl.ds(z_lo, n_rows), pl.ds(st["col"], COLS)]
                + st["rs"][pl.ds(stage_off, n_rows), :].astype(jnp.float32)
            )

        for st in streams:
            b = phi(st["order"][0])
            send_lo = (1 - b) * HALVES[0]
            st["keep0"] = b * HALVES[0]
            matmul_piece(send_lo, st["col"])
            st["op"] = start_rs(st, 0, send_lo, HALVES[0])
        for st in streams:
            matmul_piece(st["keep0"], st["col"])
            st["lo"] = st["keep0"]

        for s in range(1, 5):
            half = HALVES[s]
            for st in streams:
                st["op"].wait()
                b = phi(st["order"][s])
                send_sub = (1 - b) * half
                keep_sub = b * half
                send_lo = st["lo"] + send_sub
                add_recv(st, send_lo, OFFS[s - 1] + send_sub, half)
                st["op"] = start_rs(st, s, send_lo, half)
                st["pend"] = (st["lo"] + keep_sub, OFFS[s - 1] + keep_sub)
                st["lo"] = st["lo"] + keep_sub
            for st in streams:
                add_recv(st, st["pend"][0], st["pend"][1], half)
        for st in streams:
            st["op"].wait()
            add_recv(st, st["lo"], OFFS[4], HALVES[4])

        for st in streams:
            own = _silu_f32(
                z_ref[pl.ds(st["lo"], 48), pl.ds(st["col"], COLS)]
            )
            out_ref[pl.ds(st["lo"], 48), pl.ds(st["col"], COLS)] = own
            st["ag"][pl.ds(st["lo"], 48), :] = own.astype(jnp.bfloat16)

        for k in range(5):
            s = 4 - k
            half = HALVES[s]
            for st in streams:
                if k > 0:
                    st["op"].wait()
                b = phi(st["order"][s])
                op = rdma(
                    st, s, 5 + k,
                    st["ag"].at[pl.ds(st["lo"], half), :],
                    st["ag"].at[pl.ds(st["lo"], half), :],
                )
                op.start()
                if k > 0:
                    plo, phalf = st["pend"]
                    out_ref[pl.ds(plo, phalf), pl.ds(st["col"], COLS)] = (
                        st["ag"][pl.ds(plo, phalf), :].astype(jnp.float32)
                    )
                st["op"] = op
                st["pend"] = (st["lo"] + (1 - 2 * b) * half, half)
                st["lo"] = st["lo"] - b * half
        for st in streams:
            st["op"].wait()
            plo, phalf = st["pend"]
            out_ref[pl.ds(plo, phalf), pl.ds(st["col"], COLS)] = (
                st["ag"][pl.ds(plo, phalf), :].astype(jnp.float32)
            )

        for m in MASKS:
            pl.semaphore_signal(
                exit_sem, inc=1, device_id=(my ^ m,),
                device_id_type=pl.DeviceIdType.MESH,
            )
        pl.semaphore_wait(exit_sem, len(MASKS))

    stream_scratch = []
    for _ in range(N_STREAMS):
        stream_scratch += [
            pltpu.VMEM((M, COLS), jnp.bfloat16),
            pltpu.VMEM((M, COLS), jnp.bfloat16),
            pltpu.VMEM((M, COLS), jnp.bfloat16),
        ]

    return pl.pallas_call(
        body,
        out_shape=jax.ShapeDtypeStruct((M, N), jnp.float32),
        in_specs=[
            pl.BlockSpec(memory_space=pltpu.VMEM),
            pl.BlockSpec(memory_space=pltpu.VMEM),
        ],
        out_specs=pl.BlockSpec(memory_space=pltpu.VMEM),
        scratch_shapes=[
            pltpu.VMEM((M, N), jnp.float32),
            *stream_scratch,
            pltpu.SemaphoreType.DMA((10 * N_STREAMS,)),
            pltpu.SemaphoreType.DMA((10 * N_STREAMS,)),
            pltpu.SemaphoreType.REGULAR,
        ],
        compiler_params=pltpu.CompilerParams(collective_id=0),
    )(A, B)
